# baseline (speedup 1.0000x reference)
import jax
import jax.numpy as jnp
from jax.experimental import pallas as pl


def _dot_t(a, b):
    return jax.lax.dot_general(
        a, b, dimension_numbers=(((0,), (0,)), ((), ())),
        preferred_element_type=jnp.float32)


def _body(x_ref, aux_ref, wx_ref, waux_ref, o_ref):
    acc = _dot_t(x_ref[0], wx_ref[0])
    acc = acc + _dot_t(x_ref[1], wx_ref[1])
    acc = acc + _dot_t(aux_ref[...], waux_ref[...])
    o_ref[...] = acc


def kernel(x, node_attr_data, trainable_data, node_attr_hidden, trainable_hidden,
           edge_attr, W_data, b_data, W_hidden, b_hidden, W_edge,
           Wq, Wk, Wv, Wo, W_mlp1, W_mlp2, ln1_g, ln1_b, ln2_g, ln2_b,
           edge_index):
    B, T, Ens, G, V = x.shape
    D = W_data.shape[1]
    NA = node_attr_data.shape[1]
    TR = trainable_data.shape[1]

    xt = jnp.transpose(x.reshape(T, G, V), (0, 2, 1))
    Wx = W_data[:T * V].reshape(T, V, D)

    KA = NA + TR + 1
    aux = jnp.concatenate(
        [node_attr_data, trainable_data,
         jnp.ones((G, 1), jnp.float32),
         jnp.zeros((G, 16 - KA), jnp.float32)], axis=1).T
    Waux = jnp.concatenate(
        [W_data[T * V:], b_data.reshape(1, D),
         jnp.zeros((16 - KA, D), jnp.float32)], axis=0)

    BM = 2048
    grid = (pl.cdiv(G, BM),)

    return pl.pallas_call(
        _body,
        grid=grid,
        in_specs=[
            pl.BlockSpec((T, V, BM), lambda i: (0, 0, i)),
            pl.BlockSpec((16, BM), lambda i: (0, i)),
            pl.BlockSpec((T, V, D), lambda i: (0, 0, 0)),
            pl.BlockSpec((16, D), lambda i: (0, 0)),
        ],
        out_specs=pl.BlockSpec((BM, D), lambda i: (i, 0)),
        out_shape=jax.ShapeDtypeStruct((G, D), jnp.float32),
    )(xt, aux, Wx, Waux)


# speedup vs baseline: 3.0759x; 3.0759x over previous
import jax
import jax.numpy as jnp
from jax.experimental import pallas as pl


def _dot_t(a, b):
    return jax.lax.dot_general(
        a, b, dimension_numbers=(((0,), (0,)), ((), ())),
        preferred_element_type=jnp.float32)


def _body(x_ref, aux_ref, wx_ref, waux_ref, o_ref):
    acc = _dot_t(x_ref[0], wx_ref[0])
    acc = acc + _dot_t(x_ref[1], wx_ref[1])
    acc = acc + _dot_t(aux_ref[...], waux_ref[...])
    o_ref[...] = acc


def kernel(x, node_attr_data, trainable_data, node_attr_hidden, trainable_hidden,
           edge_attr, W_data, b_data, W_hidden, b_hidden, W_edge,
           Wq, Wk, Wv, Wo, W_mlp1, W_mlp2, ln1_g, ln1_b, ln2_g, ln2_b,
           edge_index):
    B, T, Ens, G, V = x.shape
    D = W_data.shape[1]
    NA = node_attr_data.shape[1]
    TR = trainable_data.shape[1]

    eps = jnp.float32(0.0) * b_data[0]
    xt = jnp.transpose(x.reshape(T, G, V), (0, 2, 1)) + eps
    Wx = W_data[:T * V].reshape(T, V, D)

    KA = NA + TR + 1
    aux = jnp.concatenate(
        [node_attr_data, trainable_data,
         jnp.ones((G, 1), jnp.float32),
         jnp.zeros((G, 16 - KA), jnp.float32)], axis=1).T
    Waux = jnp.concatenate(
        [W_data[T * V:], b_data.reshape(1, D),
         jnp.zeros((16 - KA, D), jnp.float32)], axis=0)

    BM = 2048
    grid = (pl.cdiv(G, BM),)

    return pl.pallas_call(
        _body,
        grid=grid,
        in_specs=[
            pl.BlockSpec((T, V, BM), lambda i: (0, 0, i)),
            pl.BlockSpec((16, BM), lambda i: (0, i)),
            pl.BlockSpec((T, V, D), lambda i: (0, 0, 0)),
            pl.BlockSpec((16, D), lambda i: (0, 0)),
        ],
        out_specs=pl.BlockSpec((BM, D), lambda i: (i, 0)),
        out_shape=jax.ShapeDtypeStruct((G, D), jnp.float32),
    )(xt, aux, Wx, Waux)
